# Initial kernel scaffold; baseline (speedup 1.0000x reference)
import jax
import jax.numpy as jnp
from jax.experimental import pallas as pl


def kernel(X, force):
    raise NotImplementedError("write your pallas kernel here")


# speedup vs baseline: 6.0781x; 6.0781x over previous
import functools

import jax
import jax.numpy as jnp
from jax import lax
from jax.experimental import pallas as pl
from jax.experimental.pallas import tpu as pltpu
from jax.experimental.pallas import tpu_sc as plsc

_N = 256
_DT = 0.05
_LANES = 16


def _make_kernel(total_elems: int, num_workers: int):
    chunk = total_elems // num_workers
    assert chunk % (2 * _LANES) == 0 and chunk * num_workers == total_elems
    steps = chunk // (2 * _LANES)
    mesh = plsc.VectorSubcoreMesh(core_axis_name="c", subcore_axis_name="s")
    nc = mesh.num_cores

    @functools.partial(
        pl.kernel,
        out_type=jax.ShapeDtypeStruct((total_elems,), jnp.float32),
        mesh=mesh,
        scratch_types=[
            pltpu.VMEM((chunk,), jnp.float32),
            pltpu.VMEM((_N,), jnp.float32),
        ],
        compiler_params=pltpu.CompilerParams(needs_layout_passes=False),
    )
    def run(x_hbm, f_hbm, out_hbm, buf, ftab):
        wid = lax.axis_index("s") * nc + lax.axis_index("c")
        base = wid * chunk
        pltpu.sync_copy(f_hbm, ftab)
        pltpu.sync_copy(x_hbm.at[pl.ds(base, chunk)], buf)

        even0 = lax.iota(jnp.int32, _LANES) * 2

        def step(i, carry):
            ev = even0 + i * (2 * _LANES)
            xg = plsc.load_gather(buf, [ev])
            t = xg * 128.0 + 128.5
            t = jnp.minimum(jnp.maximum(t, 0.0), 255.0)
            idx = t.astype(jnp.int32)
            fv = plsc.load_gather(ftab, [idx])
            plsc.addupdate_scatter(buf, [ev + 1], fv * _DT)
            return carry

        lax.fori_loop(0, steps, step, 0)
        pltpu.sync_copy(buf, out_hbm.at[pl.ds(base, chunk)])

    return run


def kernel(X, force):
    b = X.shape[0]
    flat = jnp.reshape(X, (2 * b,)).astype(jnp.float32)
    out = _make_kernel(2 * b, 32)(flat, force.astype(jnp.float32))
    return jnp.reshape(out, (b, 2))
